# baseline (speedup 1.0000x reference)
import dataclasses
import functools

import jax
import jax.numpy as jnp
from jax import lax
from jax.experimental import pallas as pl
from jax.experimental.pallas import tpu as pltpu
from jax.experimental.pallas import tpu_sc as plsc

B, T, D, F, E, TOPK = 1, 2048, 1024, 1408, 8, 2
N = T * TOPK
BM = 128
NT = N // BM
NS = NT + E - 1

SC_CORES = 2
SC_SUBCORES = 16
SC_WORKERS = SC_CORES * SC_SUBCORES
def _sc_mesh():
  return plsc.VectorSubcoreMesh(core_axis_name="c", subcore_axis_name="s",
                                num_cores=SC_CORES, num_subcores=SC_SUBCORES)


def _sc_params():
  cp = pltpu.CompilerParams()
  if "needs_layout_passes" in pltpu.CompilerParams.__dataclass_fields__:
    cp = dataclasses.replace(cp, needs_layout_passes=False)
  return cp

GATHER_CHUNK = 8


def _sc_dispatch(x2, w_flat, pe, po, pos):
  tok_per_w = T // SC_WORKERS
  pair_per_w = 2 * tok_per_w

  @functools.partial(
      pl.kernel,
      out_type=(jax.ShapeDtypeStruct((N, D), jnp.float32),
                jax.ShapeDtypeStruct((N, 128), jnp.float32)),
      mesh=_sc_mesh(),
      compiler_params=_sc_params(),
      scratch_types=[
          pltpu.VMEM((tok_per_w, D), jnp.float32),
          pltpu.VMEM((tok_per_w,), jnp.int32),
          pltpu.VMEM((tok_per_w,), jnp.int32),
          pltpu.VMEM((pair_per_w,), jnp.int32),
          pltpu.VMEM((pair_per_w,), jnp.float32),
          pltpu.VMEM((pair_per_w, 128), jnp.float32),
          pltpu.SemaphoreType.DMA,
          pltpu.SemaphoreType.DMA,
          pltpu.SemaphoreType.DMA,
          pltpu.SemaphoreType.DMA,
      ],
  )
  def dispatch(x_hbm, w_hbm, pe_hbm, po_hbm, pos_hbm, xs_hbm, ws_hbm,
               xbuf, pe_v, po_v, pw_v, wb_v, wrows, si, s0, s1, sw):
    wid = lax.axis_index("s") * SC_CORES + lax.axis_index("c")
    tbase = wid * tok_per_w
    pbase = wid * pair_per_w
    cpx = pltpu.async_copy(x_hbm.at[pl.ds(tbase, tok_per_w)], xbuf, si)
    pltpu.sync_copy(pe_hbm.at[pl.ds(tbase, tok_per_w)], pe_v)
    pltpu.sync_copy(po_hbm.at[pl.ds(tbase, tok_per_w)], po_v)
    pltpu.sync_copy(pos_hbm.at[pl.ds(pbase, pair_per_w)], pw_v)
    pltpu.sync_copy(w_hbm.at[pl.ds(pbase, pair_per_w)], wb_v)

    @pl.loop(0, pair_per_w // 16)
    def _(c):
      rows = jax.lax.iota(jnp.int32, 16) + c * 16
      vals = wb_v.at[pl.ds(pl.multiple_of(c * 16, 16), 16)][...]
      plsc.store_scatter(wrows, [rows, rows * 0], vals)

    cpw = pltpu.async_copy(wrows, ws_hbm.at[pw_v], sw)
    cpx.wait()
    cp0 = pltpu.async_copy(xbuf, xs_hbm.at[pe_v], s0)
    cp1 = pltpu.async_copy(xbuf, xs_hbm.at[po_v], s1)
    cpw.wait()
    cp0.wait()
    cp1.wait()

  return dispatch(x2, w_flat, pe, po, pos)


def _sc_combine(yw, p0, p1):
  toks_per_w = T // SC_WORKERS
  ch = 16
  n_chunks = toks_per_w // ch

  @functools.partial(
      pl.kernel,
      out_type=jax.ShapeDtypeStruct((T, D), jnp.float32),
      mesh=_sc_mesh(),
      scratch_types=[
          pltpu.VMEM((toks_per_w,), jnp.int32),
          pltpu.VMEM((toks_per_w,), jnp.int32),
          pltpu.VMEM((ch, D), jnp.float32),
          pltpu.VMEM((ch, D), jnp.float32),
          pltpu.VMEM((ch, D), jnp.float32),
          pltpu.VMEM((ch, D), jnp.float32),
          pltpu.SemaphoreType.DMA,
          pltpu.SemaphoreType.DMA,
          pltpu.SemaphoreType.DMA,
          pltpu.SemaphoreType.DMA,
      ],
  )
  def combine(y_hbm, p0_hbm, p1_hbm, out_hbm, i0_v, i1_v,
              a0, b0, a1, b1, sa0, sb0, sa1, sb1):
    wid = lax.axis_index("s") * SC_CORES + lax.axis_index("c")
    base = wid * toks_per_w
    pltpu.sync_copy(p0_hbm.at[pl.ds(base, toks_per_w)], i0_v)
    pltpu.sync_copy(p1_hbm.at[pl.ds(base, toks_per_w)], i1_v)

    def start(c, abuf, bbuf, sa, sb):
      sl = pl.ds(pl.multiple_of(c * ch, ch), ch)
      pltpu.async_copy(y_hbm.at[i0_v.at[sl]], abuf, sa)
      pltpu.async_copy(y_hbm.at[i1_v.at[sl]], bbuf, sb)

    def finish(c, abuf, bbuf, sa, sb):
      pltpu.make_async_copy(y_hbm.at[pl.ds(0, ch)], abuf, sa).wait()
      pltpu.make_async_copy(y_hbm.at[pl.ds(0, ch)], bbuf, sb).wait()

      @pl.loop(0, ch)
      def _(r):
        @pl.loop(0, D // 16, step=4)
        def _(k):
          for kk in range(4):
            sl2 = pl.ds((k + kk) * 16, 16)
            abuf.at[r, sl2][...] = abuf.at[r, sl2][...] + bbuf.at[r, sl2][...]

      pltpu.sync_copy(abuf, out_hbm.at[pl.ds(base + c * ch, ch)])

    start(0, a0, b0, sa0, sb0)

    @pl.loop(0, n_chunks, step=2)
    def _(c):
      @pl.when(c + 1 < n_chunks)
      def _():
        start(c + 1, a1, b1, sa1, sb1)
      finish(c, a0, b0, sa0, sb0)

      @pl.when(c + 1 < n_chunks)
      def _():
        @pl.when(c + 2 < n_chunks)
        def _():
          start(c + 2, a0, b0, sa0, sb0)
        finish(c + 1, a1, b1, sa1, sb1)

  return combine(yw, p0, p1)


def _ffn_body(se_ref, sm_ref, slo_ref, shi_ref, xs_ref, g_ref, u_ref, d_ref,
              w_ref, out_ref):
  s = pl.program_id(0)
  lo = slo_ref[s]
  hi = shi_ref[s]

  @pl.when(hi > lo)
  def _():
    xb = xs_ref[...].astype(jnp.bfloat16)
    g = jnp.dot(xb, g_ref[0],
                preferred_element_type=jnp.float32)
    u = jnp.dot(xb, u_ref[0],
                preferred_element_type=jnp.float32)
    h = (g * jax.nn.sigmoid(g)) * u
    h = h * w_ref[0][:, 0:1]
    y = jnp.dot(h.astype(jnp.bfloat16), d_ref[0],
                preferred_element_type=jnp.float32)
    rows = lax.broadcasted_iota(jnp.int32, (BM, D), 0)
    keep = (rows >= lo) & (rows < hi)
    out_ref[...] = jnp.where(keep, y, out_ref[...])


def _tc_grouped_ffn(xs, gate_w, up_w, down_w, w_tile, sched):
  se, sm, slo, shi = sched

  def _m_map(s, se_, sm_, slo_, shi_):
    return (sm_[s], 0)

  def _e_map(s, se_, sm_, slo_, shi_):
    return (se_[s], 0, 0)

  grid_spec = pltpu.PrefetchScalarGridSpec(
      num_scalar_prefetch=4,
      grid=(NS,),
      in_specs=[
          pl.BlockSpec((BM, D), _m_map),
          pl.BlockSpec((1, D, F), _e_map),
          pl.BlockSpec((1, D, F), _e_map),
          pl.BlockSpec((1, F, D), _e_map),
          pl.BlockSpec((1, BM, 128),
                       lambda s, se_, sm_, slo_, shi_: (sm_[s], 0, 0)),
      ],
      out_specs=pl.BlockSpec((BM, D), _m_map),
  )
  return pl.pallas_call(
      _ffn_body,
      grid_spec=grid_spec,
      out_shape=jax.ShapeDtypeStruct((N, D), jnp.float32),
      compiler_params=pltpu.CompilerParams(
          dimension_semantics=("arbitrary",)),
  )(se, sm, slo, shi, xs, gate_w, up_w, down_w, w_tile)


def kernel(x, router_w, gate_proj, up_proj, down_proj):
  scores = jnp.einsum('BTD,DE->BTE', x, router_w).astype(jnp.float32)
  top_s, routing_idx = jax.lax.top_k(scores, TOPK)
  w_pair = jax.nn.softmax(top_s, axis=-1)

  e_flat = routing_idx[0].reshape(N).astype(jnp.int32)
  w_flat = w_pair[0].reshape(N).astype(jnp.float32)

  one_hot = (e_flat[:, None] == jnp.arange(E, dtype=jnp.int32)[None, :]
             ).astype(jnp.int32)
  csum = jnp.cumsum(one_hot, axis=0)
  counts = csum[-1]
  starts = jnp.concatenate([jnp.zeros((1,), jnp.int32),
                            jnp.cumsum(counts)[:-1].astype(jnp.int32)])
  ends = (starts + counts).astype(jnp.int32)
  rank = jnp.take_along_axis(csum, e_flat[:, None], axis=1)[:, 0] - 1
  pos = (starts[e_flat] + rank).astype(jnp.int32)
  pe = pos[0::2]
  po = pos[1::2]

  x2 = x[0]
  xs, ws = _sc_dispatch(x2, w_flat, pe, po, pos)

  st_t = starts // BM
  t_e = jnp.maximum((ends - 1) // BM - st_t + 1, 0)
  off = jnp.concatenate([jnp.zeros((1,), jnp.int32),
                         jnp.cumsum(t_e)[:-1].astype(jnp.int32)])
  s_idx = jnp.arange(NS, dtype=jnp.int32)
  ge = (s_idx[:, None] >= off[None, :])
  se = jnp.maximum(jnp.sum(ge.astype(jnp.int32), axis=1) - 1, 0)
  sm = jnp.clip(st_t[se] + (s_idx - off[se]), 0, NT - 1)
  slo = jnp.clip(starts[se] - sm * BM, 0, BM).astype(jnp.int32)
  shi = jnp.clip(ends[se] - sm * BM, 0, BM).astype(jnp.int32)

  w_tile = ws.reshape(NT, BM, 128)
  yw = _tc_grouped_ffn(xs, gate_proj.astype(jnp.bfloat16),
                       up_proj.astype(jnp.bfloat16),
                       down_proj.astype(jnp.bfloat16), w_tile,
                       (se.astype(jnp.int32), sm.astype(jnp.int32),
                        slo, shi))

  out = _sc_combine(yw, pe, po)
  return out.reshape(B, T, D)


# speedup vs baseline: 1.2758x; 1.2758x over previous
import dataclasses
import functools

import jax
import jax.numpy as jnp
from jax import lax
from jax.experimental import pallas as pl
from jax.experimental.pallas import tpu as pltpu
from jax.experimental.pallas import tpu_sc as plsc

B, T, D, F, E, TOPK = 1, 2048, 1024, 1408, 8, 2
N = T * TOPK
BM = 256
NT = N // BM
NS = NT + E - 1

SC_CORES = 2
SC_SUBCORES = 16
SC_WORKERS = SC_CORES * SC_SUBCORES
def _sc_mesh():
  return plsc.VectorSubcoreMesh(core_axis_name="c", subcore_axis_name="s",
                                num_cores=SC_CORES, num_subcores=SC_SUBCORES)


def _sc_params():
  cp = pltpu.CompilerParams()
  if "needs_layout_passes" in pltpu.CompilerParams.__dataclass_fields__:
    cp = dataclasses.replace(cp, needs_layout_passes=False)
  return cp

GATHER_CHUNK = 8


def _sc_dispatch(x2, w_flat, pe, po, pos):
  tok_per_w = T // SC_WORKERS
  pair_per_w = 2 * tok_per_w

  @functools.partial(
      pl.kernel,
      out_type=(jax.ShapeDtypeStruct((N, D), jnp.float32),
                jax.ShapeDtypeStruct((N, 128), jnp.float32)),
      mesh=_sc_mesh(),
      compiler_params=_sc_params(),
      scratch_types=[
          pltpu.VMEM((tok_per_w, D), jnp.float32),
          pltpu.VMEM((tok_per_w,), jnp.int32),
          pltpu.VMEM((tok_per_w,), jnp.int32),
          pltpu.VMEM((pair_per_w,), jnp.int32),
          pltpu.VMEM((pair_per_w,), jnp.float32),
          pltpu.VMEM((pair_per_w, 128), jnp.float32),
          pltpu.SemaphoreType.DMA,
          pltpu.SemaphoreType.DMA,
          pltpu.SemaphoreType.DMA,
          pltpu.SemaphoreType.DMA,
      ],
  )
  def dispatch(x_hbm, w_hbm, pe_hbm, po_hbm, pos_hbm, xs_hbm, ws_hbm,
               xbuf, pe_v, po_v, pw_v, wb_v, wrows, si, s0, s1, sw):
    wid = lax.axis_index("s") * SC_CORES + lax.axis_index("c")
    tbase = wid * tok_per_w
    pbase = wid * pair_per_w
    cpx = pltpu.async_copy(x_hbm.at[pl.ds(tbase, tok_per_w)], xbuf, si)
    pltpu.sync_copy(pe_hbm.at[pl.ds(tbase, tok_per_w)], pe_v)
    pltpu.sync_copy(po_hbm.at[pl.ds(tbase, tok_per_w)], po_v)
    pltpu.sync_copy(pos_hbm.at[pl.ds(pbase, pair_per_w)], pw_v)
    pltpu.sync_copy(w_hbm.at[pl.ds(pbase, pair_per_w)], wb_v)

    @pl.loop(0, pair_per_w // 16)
    def _(c):
      rows = jax.lax.iota(jnp.int32, 16) + c * 16
      vals = wb_v.at[pl.ds(pl.multiple_of(c * 16, 16), 16)][...]
      plsc.store_scatter(wrows, [rows, rows * 0], vals)

    cpw = pltpu.async_copy(wrows, ws_hbm.at[pw_v], sw)
    cpx.wait()
    cp0 = pltpu.async_copy(xbuf, xs_hbm.at[pe_v], s0)
    cp1 = pltpu.async_copy(xbuf, xs_hbm.at[po_v], s1)
    cpw.wait()
    cp0.wait()
    cp1.wait()

  return dispatch(x2, w_flat, pe, po, pos)


def _sc_combine(yw, p0, p1):
  toks_per_w = T // SC_WORKERS
  ch = 16
  n_chunks = toks_per_w // ch

  @functools.partial(
      pl.kernel,
      out_type=jax.ShapeDtypeStruct((T, D), jnp.float32),
      mesh=_sc_mesh(),
      scratch_types=[
          pltpu.VMEM((toks_per_w,), jnp.int32),
          pltpu.VMEM((toks_per_w,), jnp.int32),
          pltpu.VMEM((ch, D), jnp.float32),
          pltpu.VMEM((ch, D), jnp.float32),
          pltpu.VMEM((ch, D), jnp.float32),
          pltpu.VMEM((ch, D), jnp.float32),
          pltpu.SemaphoreType.DMA,
          pltpu.SemaphoreType.DMA,
          pltpu.SemaphoreType.DMA,
          pltpu.SemaphoreType.DMA,
      ],
  )
  def combine(y_hbm, p0_hbm, p1_hbm, out_hbm, i0_v, i1_v,
              a0, b0, a1, b1, sa0, sb0, sa1, sb1):
    wid = lax.axis_index("s") * SC_CORES + lax.axis_index("c")
    base = wid * toks_per_w
    pltpu.sync_copy(p0_hbm.at[pl.ds(base, toks_per_w)], i0_v)
    pltpu.sync_copy(p1_hbm.at[pl.ds(base, toks_per_w)], i1_v)

    def start(c, abuf, bbuf, sa, sb):
      sl = pl.ds(pl.multiple_of(c * ch, ch), ch)
      pltpu.async_copy(y_hbm.at[i0_v.at[sl]], abuf, sa)
      pltpu.async_copy(y_hbm.at[i1_v.at[sl]], bbuf, sb)

    def finish(c, abuf, bbuf, sa, sb):
      pltpu.make_async_copy(y_hbm.at[pl.ds(0, ch)], abuf, sa).wait()
      pltpu.make_async_copy(y_hbm.at[pl.ds(0, ch)], bbuf, sb).wait()

      @pl.loop(0, ch)
      def _(r):
        @pl.loop(0, D // 16, step=4)
        def _(k):
          for kk in range(4):
            sl2 = pl.ds((k + kk) * 16, 16)
            abuf.at[r, sl2][...] = abuf.at[r, sl2][...] + bbuf.at[r, sl2][...]

      pltpu.sync_copy(abuf, out_hbm.at[pl.ds(base + c * ch, ch)])

    start(0, a0, b0, sa0, sb0)

    @pl.loop(0, n_chunks, step=2)
    def _(c):
      @pl.when(c + 1 < n_chunks)
      def _():
        start(c + 1, a1, b1, sa1, sb1)
      finish(c, a0, b0, sa0, sb0)

      @pl.when(c + 1 < n_chunks)
      def _():
        @pl.when(c + 2 < n_chunks)
        def _():
          start(c + 2, a0, b0, sa0, sb0)
        finish(c + 1, a1, b1, sa1, sb1)

  return combine(yw, p0, p1)


def _ffn_body(se_ref, sm_ref, slo_ref, shi_ref, xs_ref, g_ref, u_ref, d_ref,
              w_ref, out_ref, gq, uq, dq, last_ref):
  s = pl.program_id(0)
  e_sel = se_ref[s]
  lo = slo_ref[s]
  hi = shi_ref[s]

  @pl.when((s == 0) | (e_sel != last_ref[0]))
  def _():
    gq[...] = g_ref[0].astype(jnp.bfloat16)
    uq[...] = u_ref[0].astype(jnp.bfloat16)
    dq[...] = d_ref[0].astype(jnp.bfloat16)

  last_ref[0] = e_sel

  @pl.when(hi > lo)
  def _():
    xb = xs_ref[...].astype(jnp.bfloat16)
    g = jnp.dot(xb, gq[...],
                preferred_element_type=jnp.float32)
    u = jnp.dot(xb, uq[...],
                preferred_element_type=jnp.float32)
    h = (g * jax.nn.sigmoid(g)) * u
    h = h * w_ref[0][:, 0:1]
    y = jnp.dot(h.astype(jnp.bfloat16), dq[...],
                preferred_element_type=jnp.float32)
    rows = lax.broadcasted_iota(jnp.int32, (BM, D), 0)
    keep = (rows >= lo) & (rows < hi)
    out_ref[...] = jnp.where(keep, y, out_ref[...])


def _tc_grouped_ffn(xs, gate_w, up_w, down_w, w_tile, sched):
  se, sm, slo, shi = sched

  def _m_map(s, se_, sm_, slo_, shi_):
    return (sm_[s], 0)

  def _e_map(s, se_, sm_, slo_, shi_):
    return (se_[s], 0, 0)

  grid_spec = pltpu.PrefetchScalarGridSpec(
      num_scalar_prefetch=4,
      grid=(NS,),
      in_specs=[
          pl.BlockSpec((BM, D), _m_map),
          pl.BlockSpec((1, D, F), _e_map),
          pl.BlockSpec((1, D, F), _e_map),
          pl.BlockSpec((1, F, D), _e_map),
          pl.BlockSpec((1, BM, 128),
                       lambda s, se_, sm_, slo_, shi_: (sm_[s], 0, 0)),
      ],
      out_specs=pl.BlockSpec((BM, D), _m_map),
      scratch_shapes=[
          pltpu.VMEM((D, F), jnp.bfloat16),
          pltpu.VMEM((D, F), jnp.bfloat16),
          pltpu.VMEM((F, D), jnp.bfloat16),
          pltpu.SMEM((1,), jnp.int32),
      ],
  )
  return pl.pallas_call(
      _ffn_body,
      grid_spec=grid_spec,
      out_shape=jax.ShapeDtypeStruct((N, D), jnp.float32),
      compiler_params=pltpu.CompilerParams(
          dimension_semantics=("arbitrary",)),
  )(se, sm, slo, shi, xs, gate_w, up_w, down_w, w_tile)


def kernel(x, router_w, gate_proj, up_proj, down_proj):
  scores = jnp.einsum('BTD,DE->BTE', x, router_w).astype(jnp.float32)
  top_s, routing_idx = jax.lax.top_k(scores, TOPK)
  w_pair = jax.nn.softmax(top_s, axis=-1)

  e_flat = routing_idx[0].reshape(N).astype(jnp.int32)
  w_flat = w_pair[0].reshape(N).astype(jnp.float32)

  one_hot = (e_flat[:, None] == jnp.arange(E, dtype=jnp.int32)[None, :]
             ).astype(jnp.int32)
  csum = jnp.cumsum(one_hot, axis=0)
  counts = csum[-1]
  starts = jnp.concatenate([jnp.zeros((1,), jnp.int32),
                            jnp.cumsum(counts)[:-1].astype(jnp.int32)])
  ends = (starts + counts).astype(jnp.int32)
  rank = jnp.take_along_axis(csum, e_flat[:, None], axis=1)[:, 0] - 1
  pos = (starts[e_flat] + rank).astype(jnp.int32)
  pe = pos[0::2]
  po = pos[1::2]

  x2 = x[0]
  xs, ws = _sc_dispatch(x2, w_flat, pe, po, pos)

  st_t = starts // BM
  t_e = jnp.maximum((ends - 1) // BM - st_t + 1, 0)
  off = jnp.concatenate([jnp.zeros((1,), jnp.int32),
                         jnp.cumsum(t_e)[:-1].astype(jnp.int32)])
  s_idx = jnp.arange(NS, dtype=jnp.int32)
  ge = (s_idx[:, None] >= off[None, :])
  se = jnp.maximum(jnp.sum(ge.astype(jnp.int32), axis=1) - 1, 0)
  sm = jnp.clip(st_t[se] + (s_idx - off[se]), 0, NT - 1)
  slo = jnp.clip(starts[se] - sm * BM, 0, BM).astype(jnp.int32)
  shi = jnp.clip(ends[se] - sm * BM, 0, BM).astype(jnp.int32)

  w_tile = ws.reshape(NT, BM, 128)
  yw = _tc_grouped_ffn(xs, gate_proj, up_proj, down_proj, w_tile,
                       (se.astype(jnp.int32), sm.astype(jnp.int32),
                        slo, shi))

  out = _sc_combine(yw, pe, po)
  return out.reshape(B, T, D)
